# baseline (speedup 1.0000x reference)
import jax
import jax.numpy as jnp
from jax import lax
from jax.experimental import pallas as pl
from jax.experimental.pallas import tpu as pltpu
from jax.experimental.pallas import tpu_sc as plsc

B, S, H = 1, 2048, 1024
E = 8
NSLOT = 9
FF = 2048
T = B * S

BLK = 256
PR = 2 * T + E * BLK
NBR = PR // BLK
SINK = PR
PPAD = PR + 16

NC, NS, L = 2, 16, 16
NW = NC * NS
TPW = T // NW

_F32 = jnp.float32
_I32 = jnp.int32


def _router_body(x_ref, rw_ref, out_ref, meta_ref, z_ref):
    x = x_ref[...]
    rw = rw_ref[...]
    logits = lax.dot_general(x, rw, (((1,), (1,)), ((), ())),
                             preferred_element_type=_F32)
    lane = lax.broadcasted_iota(_I32, (T, 16), 1)
    valid = lane < NSLOT
    masked = jnp.where(valid, logits, -1e30)
    m = jnp.max(masked, axis=1, keepdims=True)
    p = jnp.where(valid, jnp.exp(masked - m), 0.0)
    probs = p / jnp.sum(p, axis=1, keepdims=True)
    m1 = jnp.max(probs, axis=1, keepdims=True)
    c1 = jnp.min(jnp.where(probs == m1, lane, 99), axis=1, keepdims=True)
    probs2 = jnp.where(lane == c1, -1.0, probs)
    m2 = jnp.max(probs2, axis=1, keepdims=True)
    c2 = jnp.min(jnp.where(probs2 == m2, lane, 99), axis=1, keepdims=True)
    sw = m1 + m2 + 1e-6
    w1 = m1 / sw
    w2 = m2 / sw
    z_ref[0, 0] = jnp.sum(logits * logits)

    mm1 = (lane == c1).astype(_F32)
    mm2 = (lane == c2).astype(_F32)
    mm = mm1 + mm2
    r = lax.broadcasted_iota(_I32, (T, T), 0)
    cc = lax.broadcasted_iota(_I32, (T, T), 1)
    lt = (cc < r).astype(_F32)
    cex = lax.dot_general(lt, mm, (((1,), (0,)), ((), ())),
                          preferred_element_type=_F32)
    counts = cex[T - 1:T, :] + mm[T - 1:T, :]

    lane_r = lane[0:1, :]
    cnt1 = counts + (lane_r == 0).astype(_F32)
    padded = jnp.where(lane_r < E,
                       jnp.ceil(cnt1 / BLK) * BLK, 0.0)
    ltl = (lane[0:16, :] < lax.broadcasted_iota(_I32, (16, 16), 0))
    start = lax.dot_general(padded, ltl.astype(_F32),
                            (((1,), (0,)), ((), ())),
                            preferred_element_type=_F32)

    pos_base = start + cex
    pos1 = jnp.sum(pos_base * mm1, axis=1, keepdims=True)
    pos2 = jnp.sum(pos_base * mm2, axis=1, keepdims=True)
    dummy = jnp.sum((start + counts) * (lane_r == 0).astype(_F32))
    sinkf = float(SINK)
    is_r1 = c1 < E
    is_r2 = c2 < E
    cpos1 = jnp.where(is_r1, pos1, dummy)
    cpos2 = jnp.where(is_r2, pos2, dummy)
    idx1 = jnp.where(is_r1, pos1, sinkf)
    idx2 = jnp.where(is_r2, pos2, sinkf)

    lane128 = lax.broadcasted_iota(_I32, (T, 128), 1)
    def bc(v):
        return jnp.broadcast_to(v, (T, 128))
    out = jnp.where(lane128 == 0, bc(c1.astype(_F32)),
          jnp.where(lane128 == 1, bc(c2.astype(_F32)),
          jnp.where(lane128 == 2, bc(w1),
          jnp.where(lane128 == 3, bc(w2),
          jnp.where(lane128 == 4, bc(cpos1),
          jnp.where(lane128 == 5, bc(cpos2),
          jnp.where(lane128 == 6, bc(idx1),
          jnp.where(lane128 == 7, bc(idx2), 0.0))))))))
    out_ref[...] = out

    bl = lax.broadcasted_iota(_I32, (16, 128), 1).astype(_F32) * BLK
    startc = jnp.broadcast_to(jnp.transpose(start), (16, 128))
    paddedc = jnp.broadcast_to(jnp.transpose(padded), (16, 128))
    lane16c = lax.broadcasted_iota(_I32, (16, 128), 0)
    inseg = ((bl >= startc) & (bl < startc + paddedc)
             & (lane16c < E)).astype(_F32)
    eidx = lane16c.astype(_F32) * inseg
    ones16 = jnp.ones((1, 16), _F32)
    bexp = lax.dot_general(ones16, eidx, (((1,), (0,)), ((), ())),
                           preferred_element_type=_F32)
    bact = lax.dot_general(ones16, inseg, (((1,), (0,)), ((), ())),
                           preferred_element_type=_F32)
    eye = (lax.broadcasted_iota(_I32, (16, 128), 0)
           == lax.broadcasted_iota(_I32, (16, 128), 1)).astype(_F32)
    counts128 = lax.dot_general(counts, eye, (((1,), (0,)), ((), ())),
                                preferred_element_type=_F32)
    row8 = lax.broadcasted_iota(_I32, (8, 128), 0)
    meta = jnp.where(row8 == 0, jnp.broadcast_to(bexp, (8, 128)),
           jnp.where(row8 == 1, jnp.broadcast_to(bact, (8, 128)),
           jnp.where(row8 == 2, jnp.broadcast_to(counts128, (8, 128)),
           jnp.where(row8 == 3,
                     jnp.where(lane128[0:8, :] == 0, dummy, sinkf), 0.0))))
    meta_ref[...] = meta


def _run_router(x2d, router_w):
    rw16 = jnp.zeros((16, H), _F32).at[:NSLOT].set(router_w)
    return pl.pallas_call(
        _router_body,
        out_shape=[jax.ShapeDtypeStruct((T, 128), _F32),
                   jax.ShapeDtypeStruct((8, 128), _F32),
                   jax.ShapeDtypeStruct((1, 1), _F32)],
        out_specs=[pl.BlockSpec(memory_space=pltpu.VMEM),
                   pl.BlockSpec(memory_space=pltpu.VMEM),
                   pl.BlockSpec(memory_space=pltpu.SMEM)],
    )(x2d, rw16)


CSZ = 16


def _dispatch_body(x_hbm, idx1_hbm, idx2_hbm, w1_hbm, w2_hbm, dmy_hbm,
                   z_hbm, xd_hbm, wrow_hbm,
                   i1r, i2r, w1v, w2v, rva, rvb, rvc, rvd, dmyv, zrow, zw,
                   sem, semw):
    wid = lax.axis_index("s") * NC + lax.axis_index("c")
    t0 = wid * TPW
    nch = TPW // CSZ
    pltpu.sync_copy(idx1_hbm.at[pl.ds(wid * nch, nch)], i1r)
    pltpu.sync_copy(idx2_hbm.at[pl.ds(wid * nch, nch)], i2r)
    pltpu.sync_copy(w1_hbm.at[pl.ds(t0, TPW)], w1v)
    pltpu.sync_copy(w2_hbm.at[pl.ds(t0, TPW)], w2v)
    rvs = (rva, rvb, rvc, rvd)
    pend = []
    for j in range(nch):
        pltpu.sync_copy(x_hbm.at[pl.ds(t0 + j * CSZ, CSZ)], rvs[j])
        pend.append(pltpu.async_copy(rvs[j], xd_hbm.at[i1r.at[j]], sem))
        pend.append(pltpu.async_copy(rvs[j], xd_hbm.at[i2r.at[j]], sem))
        pend.append(pltpu.async_copy(w1v.at[pl.ds(j * CSZ, CSZ)],
                                     wrow_hbm.at[i1r.at[j]], sem))
        pend.append(pltpu.async_copy(w2v.at[pl.ds(j * CSZ, CSZ)],
                                     wrow_hbm.at[i2r.at[j]], sem))
    for cp in pend:
        cp.wait()

    @pl.when(wid == 0)
    def _dummy():
        pltpu.sync_copy(dmy_hbm, dmyv)
        pltpu.sync_copy(z_hbm, zrow)
        pltpu.sync_copy(z_hbm.at[0, pl.ds(0, L)], zw)
        pltpu.async_copy(zrow, xd_hbm.at[dmyv], semw).wait()
        pltpu.async_copy(zw, wrow_hbm.at[dmyv], semw).wait()


def _run_dispatch(x2d, idx1, idx2, w1, w2, dmy):
    mesh = plsc.VectorSubcoreMesh(core_axis_name="c", subcore_axis_name="s",
                                  num_cores=NC, num_subcores=NS)
    f = pl.kernel(
        _dispatch_body,
        out_type=[jax.ShapeDtypeStruct((PPAD, H), _F32),
                  jax.ShapeDtypeStruct((PPAD,), _F32)],
        mesh=mesh,
        compiler_params=pltpu.CompilerParams(needs_layout_passes=False),
        scratch_types=[
            pltpu.VMEM((TPW // CSZ, CSZ), _I32),
            pltpu.VMEM((TPW // CSZ, CSZ), _I32),
            pltpu.VMEM((TPW,), _F32), pltpu.VMEM((TPW,), _F32),
            pltpu.VMEM((CSZ, H), _F32), pltpu.VMEM((CSZ, H), _F32),
            pltpu.VMEM((CSZ, H), _F32), pltpu.VMEM((CSZ, H), _F32),
            pltpu.VMEM((L,), _I32),
            pltpu.VMEM((L, H), _F32),
            pltpu.VMEM((L,), _F32),
            pltpu.SemaphoreType.DMA, pltpu.SemaphoreType.DMA,
        ],
    )
    return f(x2d, idx1.reshape(T // CSZ, CSZ), idx2.reshape(T // CSZ, CSZ),
             w1, w2, dmy, jnp.zeros((L, H), _F32))


def _gffn_body(bexp_ref, bact_ref, xd_ref, gw_ref, uw_ref, dw_ref, wr_ref,
               yd_ref):
    i = pl.program_id(0)

    @pl.when(bact_ref[i] == 1)
    def _():
        xb = xd_ref[...]
        g = lax.dot_general(xb, gw_ref[0], (((1,), (1,)), ((), ())),
                            preferred_element_type=_F32)
        u = lax.dot_general(xb, uw_ref[0], (((1,), (1,)), ((), ())),
                            preferred_element_type=_F32)
        h = g * lax.logistic(g) * u
        y = lax.dot_general(h, dw_ref[0], (((1,), (1,)), ((), ())),
                            preferred_element_type=_F32)
        w = jnp.transpose(wr_ref[0])
        yd_ref[...] = y * w


def _run_gffn(bexp, bact, xd, gate_w, up_w, down_w, wrow):
    wr3d = wrow[:PR].reshape(NBR, 1, BLK)
    grid_spec = pltpu.PrefetchScalarGridSpec(
        num_scalar_prefetch=2,
        grid=(NBR,),
        in_specs=[
            pl.BlockSpec((BLK, H), lambda i, be, ba: (i, 0)),
            pl.BlockSpec((1, FF, H), lambda i, be, ba: (be[i], 0, 0)),
            pl.BlockSpec((1, FF, H), lambda i, be, ba: (be[i], 0, 0)),
            pl.BlockSpec((1, H, FF), lambda i, be, ba: (be[i], 0, 0)),
            pl.BlockSpec((1, 1, BLK), lambda i, be, ba: (i, 0, 0)),
        ],
        out_specs=pl.BlockSpec((BLK, H), lambda i, be, ba: (i, 0)),
    )
    return pl.pallas_call(
        _gffn_body,
        grid_spec=grid_spec,
        out_shape=jax.ShapeDtypeStruct((PR, H), _F32),
        compiler_params=pltpu.CompilerParams(
            vmem_limit_bytes=120 * 1024 * 1024),
    )(bexp, bact, xd, gate_w, up_w, down_w, wr3d)


def _sffn_body(x_ref, gw_ref, uw_ref, dw_ref, o_ref):
    xb = x_ref[...]
    g = lax.dot_general(xb, gw_ref[...], (((1,), (1,)), ((), ())),
                        preferred_element_type=_F32)
    u = lax.dot_general(xb, uw_ref[...], (((1,), (1,)), ((), ())),
                        preferred_element_type=_F32)
    h = g * lax.logistic(g) * u
    o_ref[...] = lax.dot_general(h, dw_ref[...], (((1,), (1,)), ((), ())),
                                 preferred_element_type=_F32)


def _run_sffn(x2d, sgw, suw, sdw):
    return pl.pallas_call(
        _sffn_body,
        grid=(T // BLK,),
        in_specs=[
            pl.BlockSpec((BLK, H), lambda i: (i, 0)),
            pl.BlockSpec((FF, H), lambda i: (0, 0)),
            pl.BlockSpec((FF, H), lambda i: (0, 0)),
            pl.BlockSpec((H, FF), lambda i: (0, 0)),
        ],
        out_specs=pl.BlockSpec((BLK, H), lambda i: (i, 0)),
        out_shape=jax.ShapeDtypeStruct((T, H), _F32),
    )(x2d, sgw, suw, sdw)


def _combine_body(yds_hbm, yd_hbm, pos1_hbm, pos2_hbm, out_hbm,
                  p1all, p2all, r1a, r2a, acca, r1b, r2b, accb, sem):
    wid = lax.axis_index("s") * NC + lax.axis_index("c")
    csz = 16
    nch = TPW // csz
    pltpu.sync_copy(pos1_hbm.at[pl.ds(wid * nch, nch)], p1all)
    pltpu.sync_copy(pos2_hbm.at[pl.ds(wid * nch, nch)], p2all)
    bufs = ((r1a, r2a, acca), (r1b, r2b, accb))

    def fetch(ch, bi):
        r1, r2, acc = bufs[bi]
        t0 = wid * TPW + ch * csz
        cp1 = pltpu.async_copy(yd_hbm.at[p1all.at[ch]], r1, sem)
        cp2 = pltpu.async_copy(yd_hbm.at[p2all.at[ch]], r2, sem)
        cp3 = pltpu.async_copy(yds_hbm.at[pl.ds(t0, csz)], acc, sem)
        return (cp1, cp2, cp3)

    pend = fetch(0, 0)
    for ch in range(nch):
        bi = ch % 2
        r1, r2, acc = bufs[bi]
        for cp in pend:
            cp.wait()
        if ch + 1 < nch:
            pend = fetch(ch + 1, (ch + 1) % 2)

        def add_row(r, _):
            for k in range(H // L):
                a = (acc[r, pl.ds(k * L, L)] + r1[r, pl.ds(k * L, L)]
                     + r2[r, pl.ds(k * L, L)])
                acc[r, pl.ds(k * L, L)] = a
            return 0

        lax.fori_loop(0, csz, add_row, 0)
        t0 = wid * TPW + ch * csz
        pltpu.sync_copy(acc, out_hbm.at[pl.ds(t0, csz)])


def _run_combine(yds, yd, pos1, pos2):
    mesh = plsc.VectorSubcoreMesh(core_axis_name="c", subcore_axis_name="s",
                                  num_cores=NC, num_subcores=NS)
    f = pl.kernel(
        _combine_body,
        out_type=[jax.ShapeDtypeStruct((T, H), _F32)],
        mesh=mesh,
        compiler_params=pltpu.CompilerParams(needs_layout_passes=False),
        scratch_types=[
            pltpu.VMEM((TPW // 16, 16), _I32), pltpu.VMEM((TPW // 16, 16), _I32),
            pltpu.VMEM((16, H), _F32), pltpu.VMEM((16, H), _F32),
            pltpu.VMEM((16, H), _F32),
            pltpu.VMEM((16, H), _F32), pltpu.VMEM((16, H), _F32),
            pltpu.VMEM((16, H), _F32),
            pltpu.SemaphoreType.DMA,
        ],
    )
    return f(yds, yd, pos1.reshape(T // 16, 16), pos2.reshape(T // 16, 16))[0]


def kernel(hidden_states, router_w, gate_w, up_w, down_w,
           shared_gate_w, shared_up_w, shared_down_w):
    x2d = hidden_states.reshape(T, H)

    rtr, meta, z_sum = _run_router(x2d, router_w)
    w1 = rtr[:, 2]
    w2 = rtr[:, 3]
    cpos1 = rtr[:, 4].astype(_I32)
    cpos2 = rtr[:, 5].astype(_I32)
    idx1 = rtr[:, 6].astype(_I32)
    idx2 = rtr[:, 7].astype(_I32)
    bexp = meta[0, :NBR].astype(_I32)
    bact = meta[1, :NBR].astype(_I32)
    counts = meta[2, :16]
    dmy = meta[3, :16].astype(_I32)

    xd, wrow = _run_dispatch(x2d, idx1, idx2, w1, w2, dmy)
    yds = _run_sffn(x2d, shared_gate_w, shared_up_w, shared_down_w)
    yd = _run_gffn(bexp, bact, xd, gate_w, up_w, down_w, wrow)
    out2d = _run_combine(yds, yd, cpos1, cpos2)

    final_outputs = out2d.reshape(B, S, H)

    loads = jnp.concatenate([counts[:E], jnp.full((1,), float(T), _F32)])
    loads_norm = loads / (jnp.sum(loads) + 1e-6)
    ideal = 1.0 / NSLOT
    load_balance_loss = jnp.mean((loads_norm - ideal) ** 2)
    router_z_loss = z_sum[0, 0] / T
    total_aux_loss = 0.01 * load_balance_loss + 0.01 * router_z_loss
    return final_outputs, total_aux_loss


# speedup vs baseline: 1.2513x; 1.2513x over previous
import jax
import jax.numpy as jnp
from jax import lax
from jax.experimental import pallas as pl
from jax.experimental.pallas import tpu as pltpu
from jax.experimental.pallas import tpu_sc as plsc

B, S, H = 1, 2048, 1024
E = 8
NSLOT = 9
FF = 2048
T = B * S

BLK = 256
PR = 2 * T + E * BLK
NBR = PR // BLK
SINK = PR
PPAD = PR + 16

NC, NS, L = 2, 16, 16
NW = NC * NS
TPW = T // NW

_F32 = jnp.float32
_I32 = jnp.int32


def _router_body(x_ref, rw_ref, out_ref, meta_ref, z_ref):
    x = x_ref[...]
    rw = rw_ref[...]
    logits = lax.dot_general(x, rw, (((1,), (1,)), ((), ())),
                             preferred_element_type=_F32)
    lane = lax.broadcasted_iota(_I32, (T, 16), 1)
    valid = lane < NSLOT
    masked = jnp.where(valid, logits, -1e30)
    m = jnp.max(masked, axis=1, keepdims=True)
    p = jnp.where(valid, jnp.exp(masked - m), 0.0)
    probs = p / jnp.sum(p, axis=1, keepdims=True)
    m1 = jnp.max(probs, axis=1, keepdims=True)
    c1 = jnp.min(jnp.where(probs == m1, lane, 99), axis=1, keepdims=True)
    probs2 = jnp.where(lane == c1, -1.0, probs)
    m2 = jnp.max(probs2, axis=1, keepdims=True)
    c2 = jnp.min(jnp.where(probs2 == m2, lane, 99), axis=1, keepdims=True)
    sw = m1 + m2 + 1e-6
    w1 = m1 / sw
    w2 = m2 / sw
    z_ref[0, 0] = jnp.sum(logits * logits)

    mm1 = (lane == c1).astype(_F32)
    mm2 = (lane == c2).astype(_F32)
    mm = mm1 + mm2
    r = lax.broadcasted_iota(_I32, (T, T), 0)
    cc = lax.broadcasted_iota(_I32, (T, T), 1)
    lt = (cc < r).astype(_F32)
    cex = lax.dot_general(lt, mm, (((1,), (0,)), ((), ())),
                          preferred_element_type=_F32)
    counts = cex[T - 1:T, :] + mm[T - 1:T, :]

    lane_r = lane[0:1, :]
    cnt1 = counts + (lane_r == 0).astype(_F32)
    padded = jnp.where(lane_r < E,
                       jnp.ceil(cnt1 / BLK) * BLK, 0.0)
    ltl = (lane[0:16, :] < lax.broadcasted_iota(_I32, (16, 16), 0))
    start = lax.dot_general(padded, ltl.astype(_F32),
                            (((1,), (0,)), ((), ())),
                            preferred_element_type=_F32)

    pos_base = start + cex
    pos1 = jnp.sum(pos_base * mm1, axis=1, keepdims=True)
    pos2 = jnp.sum(pos_base * mm2, axis=1, keepdims=True)
    dummy = jnp.sum((start + counts) * (lane_r == 0).astype(_F32))
    sinkf = float(SINK)
    is_r1 = c1 < E
    is_r2 = c2 < E
    cpos1 = jnp.where(is_r1, pos1, dummy)
    cpos2 = jnp.where(is_r2, pos2, dummy)
    idx1 = jnp.where(is_r1, pos1, sinkf)
    idx2 = jnp.where(is_r2, pos2, sinkf)

    lane128 = lax.broadcasted_iota(_I32, (T, 128), 1)
    def bc(v):
        return jnp.broadcast_to(v, (T, 128))
    out = jnp.where(lane128 == 0, bc(c1.astype(_F32)),
          jnp.where(lane128 == 1, bc(c2.astype(_F32)),
          jnp.where(lane128 == 2, bc(w1),
          jnp.where(lane128 == 3, bc(w2),
          jnp.where(lane128 == 4, bc(cpos1),
          jnp.where(lane128 == 5, bc(cpos2),
          jnp.where(lane128 == 6, bc(idx1),
          jnp.where(lane128 == 7, bc(idx2), 0.0))))))))
    out_ref[...] = out

    bl = lax.broadcasted_iota(_I32, (16, 128), 1).astype(_F32) * BLK
    startc = jnp.broadcast_to(jnp.transpose(start), (16, 128))
    paddedc = jnp.broadcast_to(jnp.transpose(padded), (16, 128))
    lane16c = lax.broadcasted_iota(_I32, (16, 128), 0)
    inseg = ((bl >= startc) & (bl < startc + paddedc)
             & (lane16c < E)).astype(_F32)
    eidx = lane16c.astype(_F32) * inseg
    ones16 = jnp.ones((1, 16), _F32)
    bexp = lax.dot_general(ones16, eidx, (((1,), (0,)), ((), ())),
                           preferred_element_type=_F32)
    bact = lax.dot_general(ones16, inseg, (((1,), (0,)), ((), ())),
                           preferred_element_type=_F32)
    eye = (lax.broadcasted_iota(_I32, (16, 128), 0)
           == lax.broadcasted_iota(_I32, (16, 128), 1)).astype(_F32)
    counts128 = lax.dot_general(counts, eye, (((1,), (0,)), ((), ())),
                                preferred_element_type=_F32)
    row8 = lax.broadcasted_iota(_I32, (8, 128), 0)
    meta = jnp.where(row8 == 0, jnp.broadcast_to(bexp, (8, 128)),
           jnp.where(row8 == 1, jnp.broadcast_to(bact, (8, 128)),
           jnp.where(row8 == 2, jnp.broadcast_to(counts128, (8, 128)),
           jnp.where(row8 == 3,
                     jnp.where(lane128[0:8, :] == 0, dummy, sinkf), 0.0))))
    meta_ref[...] = meta


def _run_router(x2d, router_w):
    rw16 = jnp.zeros((16, H), _F32).at[:NSLOT].set(router_w)
    return pl.pallas_call(
        _router_body,
        out_shape=[jax.ShapeDtypeStruct((T, 128), _F32),
                   jax.ShapeDtypeStruct((8, 128), _F32),
                   jax.ShapeDtypeStruct((1, 1), _F32)],
        out_specs=[pl.BlockSpec(memory_space=pltpu.VMEM),
                   pl.BlockSpec(memory_space=pltpu.VMEM),
                   pl.BlockSpec(memory_space=pltpu.SMEM)],
    )(x2d, rw16)


CSZ = 16


def _dispatch_body(x_hbm, idx1_hbm, idx2_hbm, dmy_hbm,
                   z_hbm, xd_hbm,
                   i1r, i2r, rva, rvb, rvc, rvd, dmyv, zrow,
                   sem, semw):
    wid = lax.axis_index("s") * NC + lax.axis_index("c")
    t0 = wid * TPW
    nch = TPW // CSZ
    pltpu.sync_copy(idx1_hbm.at[pl.ds(wid * nch, nch)], i1r)
    pltpu.sync_copy(idx2_hbm.at[pl.ds(wid * nch, nch)], i2r)
    rvs = (rva, rvb, rvc, rvd)
    pend = []
    for j in range(nch):
        pltpu.sync_copy(x_hbm.at[pl.ds(t0 + j * CSZ, CSZ)], rvs[j])
        pend.append(pltpu.async_copy(rvs[j], xd_hbm.at[i1r.at[j]], sem))
        pend.append(pltpu.async_copy(rvs[j], xd_hbm.at[i2r.at[j]], sem))
    for cp in pend:
        cp.wait()

    @pl.when(wid == 0)
    def _dummy():
        pltpu.sync_copy(dmy_hbm, dmyv)
        pltpu.sync_copy(z_hbm, zrow)
        pltpu.async_copy(zrow, xd_hbm.at[dmyv], semw).wait()


def _run_dispatch(x2d, idx1, idx2, dmy):
    mesh = plsc.VectorSubcoreMesh(core_axis_name="c", subcore_axis_name="s",
                                  num_cores=NC, num_subcores=NS)
    f = pl.kernel(
        _dispatch_body,
        out_type=[jax.ShapeDtypeStruct((PPAD, H), _F32)],
        mesh=mesh,
        compiler_params=pltpu.CompilerParams(needs_layout_passes=False),
        scratch_types=[
            pltpu.VMEM((TPW // CSZ, CSZ), _I32),
            pltpu.VMEM((TPW // CSZ, CSZ), _I32),
            pltpu.VMEM((CSZ, H), _F32), pltpu.VMEM((CSZ, H), _F32),
            pltpu.VMEM((CSZ, H), _F32), pltpu.VMEM((CSZ, H), _F32),
            pltpu.VMEM((L,), _I32),
            pltpu.VMEM((L, H), _F32),
            pltpu.SemaphoreType.DMA, pltpu.SemaphoreType.DMA,
        ],
    )
    return f(x2d, idx1.reshape(T // CSZ, CSZ), idx2.reshape(T // CSZ, CSZ),
             dmy, jnp.zeros((L, H), _F32))[0]


def _gffn_body(bexp_ref, bact_ref, xd_ref, gw_ref, uw_ref, dw_ref, yd_ref):
    i = pl.program_id(0)

    @pl.when(bact_ref[i] == 1)
    def _():
        xb = xd_ref[...]
        g = lax.dot_general(xb, gw_ref[0], (((1,), (1,)), ((), ())),
                            preferred_element_type=_F32)
        u = lax.dot_general(xb, uw_ref[0], (((1,), (1,)), ((), ())),
                            preferred_element_type=_F32)
        h = g * lax.logistic(g) * u
        yd_ref[...] = lax.dot_general(h, dw_ref[0], (((1,), (1,)), ((), ())),
                                      preferred_element_type=_F32)


def _run_gffn(bexp, bact, xd, gate_w, up_w, down_w):
    grid_spec = pltpu.PrefetchScalarGridSpec(
        num_scalar_prefetch=2,
        grid=(NBR,),
        in_specs=[
            pl.BlockSpec((BLK, H), lambda i, be, ba: (i, 0)),
            pl.BlockSpec((1, FF, H), lambda i, be, ba: (be[i], 0, 0)),
            pl.BlockSpec((1, FF, H), lambda i, be, ba: (be[i], 0, 0)),
            pl.BlockSpec((1, H, FF), lambda i, be, ba: (be[i], 0, 0)),
        ],
        out_specs=pl.BlockSpec((BLK, H), lambda i, be, ba: (i, 0)),
    )
    return pl.pallas_call(
        _gffn_body,
        grid_spec=grid_spec,
        out_shape=jax.ShapeDtypeStruct((PR, H), _F32),
        compiler_params=pltpu.CompilerParams(
            vmem_limit_bytes=120 * 1024 * 1024),
    )(bexp, bact, xd, gate_w, up_w, down_w)


def _sffn_body(x_ref, gw_ref, uw_ref, dw_ref, o_ref):
    xb = x_ref[...]
    g = lax.dot_general(xb, gw_ref[...], (((1,), (1,)), ((), ())),
                        preferred_element_type=_F32)
    u = lax.dot_general(xb, uw_ref[...], (((1,), (1,)), ((), ())),
                        preferred_element_type=_F32)
    h = g * lax.logistic(g) * u
    o_ref[...] = lax.dot_general(h, dw_ref[...], (((1,), (1,)), ((), ())),
                                 preferred_element_type=_F32)


def _run_sffn(x2d, sgw, suw, sdw):
    return pl.pallas_call(
        _sffn_body,
        grid=(T // BLK,),
        in_specs=[
            pl.BlockSpec((BLK, H), lambda i: (i, 0)),
            pl.BlockSpec((FF, H), lambda i: (0, 0)),
            pl.BlockSpec((FF, H), lambda i: (0, 0)),
            pl.BlockSpec((H, FF), lambda i: (0, 0)),
        ],
        out_specs=pl.BlockSpec((BLK, H), lambda i: (i, 0)),
        out_shape=jax.ShapeDtypeStruct((T, H), _F32),
    )(x2d, sgw, suw, sdw)


def _combine_body(yds_hbm, yd_hbm, pos1_hbm, pos2_hbm, w1_hbm, w2_hbm,
                  out_hbm,
                  p1all, p2all, w1v, w2v, r1a, r2a, acca, r1b, r2b, accb,
                  sem):
    wid = lax.axis_index("s") * NC + lax.axis_index("c")
    csz = 16
    nch = TPW // csz
    pltpu.sync_copy(pos1_hbm.at[pl.ds(wid * nch, nch)], p1all)
    pltpu.sync_copy(pos2_hbm.at[pl.ds(wid * nch, nch)], p2all)
    pltpu.sync_copy(w1_hbm.at[pl.ds(wid * TPW, TPW)], w1v)
    pltpu.sync_copy(w2_hbm.at[pl.ds(wid * TPW, TPW)], w2v)
    bufs = ((r1a, r2a, acca), (r1b, r2b, accb))

    def fetch(ch, bi):
        r1, r2, acc = bufs[bi]
        t0 = wid * TPW + ch * csz
        cp1 = pltpu.async_copy(yd_hbm.at[p1all.at[ch]], r1, sem)
        cp2 = pltpu.async_copy(yd_hbm.at[p2all.at[ch]], r2, sem)
        cp3 = pltpu.async_copy(yds_hbm.at[pl.ds(t0, csz)], acc, sem)
        return (cp1, cp2, cp3)

    pend = fetch(0, 0)
    for ch in range(nch):
        bi = ch % 2
        r1, r2, acc = bufs[bi]
        for cp in pend:
            cp.wait()
        if ch + 1 < nch:
            pend = fetch(ch + 1, (ch + 1) % 2)

        wv1 = w1v[pl.ds(ch * csz, csz)]
        wv2 = w2v[pl.ds(ch * csz, csz)]
        for r in range(csz):
            ws1 = wv1[r]
            ws2 = wv2[r]

            def add_group(k, _):
                a = (acc[r, pl.ds(k * L, L)] + ws1 * r1[r, pl.ds(k * L, L)]
                     + ws2 * r2[r, pl.ds(k * L, L)])
                acc[r, pl.ds(k * L, L)] = a
                return 0

            lax.fori_loop(0, H // L, add_group, 0)
        t0 = wid * TPW + ch * csz
        pltpu.sync_copy(acc, out_hbm.at[pl.ds(t0, csz)])


def _run_combine(yds, yd, pos1, pos2, w1, w2):
    mesh = plsc.VectorSubcoreMesh(core_axis_name="c", subcore_axis_name="s",
                                  num_cores=NC, num_subcores=NS)
    f = pl.kernel(
        _combine_body,
        out_type=[jax.ShapeDtypeStruct((T, H), _F32)],
        mesh=mesh,
        compiler_params=pltpu.CompilerParams(needs_layout_passes=False),
        scratch_types=[
            pltpu.VMEM((TPW // 16, 16), _I32), pltpu.VMEM((TPW // 16, 16), _I32),
            pltpu.VMEM((TPW,), _F32), pltpu.VMEM((TPW,), _F32),
            pltpu.VMEM((16, H), _F32), pltpu.VMEM((16, H), _F32),
            pltpu.VMEM((16, H), _F32),
            pltpu.VMEM((16, H), _F32), pltpu.VMEM((16, H), _F32),
            pltpu.VMEM((16, H), _F32),
            pltpu.SemaphoreType.DMA,
        ],
    )
    return f(yds, yd, pos1.reshape(T // 16, 16), pos2.reshape(T // 16, 16),
             w1, w2)[0]


def kernel(hidden_states, router_w, gate_w, up_w, down_w,
           shared_gate_w, shared_up_w, shared_down_w):
    x2d = hidden_states.reshape(T, H)

    rtr, meta, z_sum = _run_router(x2d, router_w)
    w1 = rtr[:, 2]
    w2 = rtr[:, 3]
    cpos1 = rtr[:, 4].astype(_I32)
    cpos2 = rtr[:, 5].astype(_I32)
    idx1 = rtr[:, 6].astype(_I32)
    idx2 = rtr[:, 7].astype(_I32)
    bexp = meta[0, :NBR].astype(_I32)
    bact = meta[1, :NBR].astype(_I32)
    counts = meta[2, :16]
    dmy = meta[3, :16].astype(_I32)

    xd = _run_dispatch(x2d, idx1, idx2, dmy)
    yds = _run_sffn(x2d, shared_gate_w, shared_up_w, shared_down_w)
    yd = _run_gffn(bexp, bact, xd, gate_w, up_w, down_w)
    out2d = _run_combine(yds, yd, cpos1, cpos2, w1, w2)

    final_outputs = out2d.reshape(B, S, H)

    loads = jnp.concatenate([counts[:E], jnp.full((1,), float(T), _F32)])
    loads_norm = loads / (jnp.sum(loads) + 1e-6)
    ideal = 1.0 / NSLOT
    load_balance_loss = jnp.mean((loads_norm - ideal) ** 2)
    router_z_loss = z_sum[0, 0] / T
    total_aux_loss = 0.01 * load_balance_loss + 0.01 * router_z_loss
    return final_outputs, total_aux_loss


# speedup vs baseline: 1.4019x; 1.1204x over previous
import jax
import jax.numpy as jnp
from jax import lax
from jax.experimental import pallas as pl
from jax.experimental.pallas import tpu as pltpu
from jax.experimental.pallas import tpu_sc as plsc

B, S, H = 1, 2048, 1024
E = 8
NSLOT = 9
FF = 2048
T = B * S

BLK = 512
PR = 2 * T + E * BLK
NBR = PR // BLK
SINK = PR
PPAD = PR + 16

NC, NS, L = 2, 16, 16
NW = NC * NS
TPW = T // NW

_F32 = jnp.float32
_I32 = jnp.int32


def _router_body(x_ref, rw_ref, out_ref, meta_ref, z_ref):
    x = x_ref[...]
    rw = rw_ref[...]
    logits = lax.dot_general(x, rw, (((1,), (1,)), ((), ())),
                             preferred_element_type=_F32)
    lane = lax.broadcasted_iota(_I32, (T, 16), 1)
    valid = lane < NSLOT
    masked = jnp.where(valid, logits, -1e30)
    m = jnp.max(masked, axis=1, keepdims=True)
    p = jnp.where(valid, jnp.exp(masked - m), 0.0)
    probs = p / jnp.sum(p, axis=1, keepdims=True)
    m1 = jnp.max(probs, axis=1, keepdims=True)
    c1 = jnp.min(jnp.where(probs == m1, lane, 99), axis=1, keepdims=True)
    probs2 = jnp.where(lane == c1, -1.0, probs)
    m2 = jnp.max(probs2, axis=1, keepdims=True)
    c2 = jnp.min(jnp.where(probs2 == m2, lane, 99), axis=1, keepdims=True)
    sw = m1 + m2 + 1e-6
    w1 = m1 / sw
    w2 = m2 / sw
    z_ref[0, 0] = jnp.sum(logits * logits)

    mm1 = (lane == c1).astype(_F32)
    mm2 = (lane == c2).astype(_F32)
    mm = mm1 + mm2
    r = lax.broadcasted_iota(_I32, (T, T), 0)
    cc = lax.broadcasted_iota(_I32, (T, T), 1)
    lt = (cc < r).astype(_F32)
    cex = lax.dot_general(lt, mm, (((1,), (0,)), ((), ())),
                          preferred_element_type=_F32)
    counts = cex[T - 1:T, :] + mm[T - 1:T, :]

    lane_r = lane[0:1, :]
    cnt1 = counts + (lane_r == 0).astype(_F32)
    padded = jnp.where(lane_r < E,
                       jnp.ceil(cnt1 / BLK) * BLK, 0.0)
    ltl = (lane[0:16, :] < lax.broadcasted_iota(_I32, (16, 16), 0))
    start = lax.dot_general(padded, ltl.astype(_F32),
                            (((1,), (0,)), ((), ())),
                            preferred_element_type=_F32)

    pos_base = start + cex
    pos1 = jnp.sum(pos_base * mm1, axis=1, keepdims=True)
    pos2 = jnp.sum(pos_base * mm2, axis=1, keepdims=True)
    dummy = jnp.sum((start + counts) * (lane_r == 0).astype(_F32))
    sinkf = float(SINK)
    is_r1 = c1 < E
    is_r2 = c2 < E
    cpos1 = jnp.where(is_r1, pos1, dummy)
    cpos2 = jnp.where(is_r2, pos2, dummy)
    idx1 = jnp.where(is_r1, pos1, sinkf)
    idx2 = jnp.where(is_r2, pos2, sinkf)

    lane128 = lax.broadcasted_iota(_I32, (T, 128), 1)
    def bc(v):
        return jnp.broadcast_to(v, (T, 128))
    out = jnp.where(lane128 == 0, bc(c1.astype(_F32)),
          jnp.where(lane128 == 1, bc(c2.astype(_F32)),
          jnp.where(lane128 == 2, bc(w1),
          jnp.where(lane128 == 3, bc(w2),
          jnp.where(lane128 == 4, bc(cpos1),
          jnp.where(lane128 == 5, bc(cpos2),
          jnp.where(lane128 == 6, bc(idx1),
          jnp.where(lane128 == 7, bc(idx2), 0.0))))))))
    out_ref[...] = out

    bl = lax.broadcasted_iota(_I32, (16, 128), 1).astype(_F32) * BLK
    startc = jnp.broadcast_to(jnp.transpose(start), (16, 128))
    paddedc = jnp.broadcast_to(jnp.transpose(padded), (16, 128))
    lane16c = lax.broadcasted_iota(_I32, (16, 128), 0)
    inseg = ((bl >= startc) & (bl < startc + paddedc)
             & (lane16c < E)).astype(_F32)
    eidx = lane16c.astype(_F32) * inseg
    ones16 = jnp.ones((1, 16), _F32)
    bexp = lax.dot_general(ones16, eidx, (((1,), (0,)), ((), ())),
                           preferred_element_type=_F32)
    bact = lax.dot_general(ones16, inseg, (((1,), (0,)), ((), ())),
                           preferred_element_type=_F32)
    eye = (lax.broadcasted_iota(_I32, (16, 128), 0)
           == lax.broadcasted_iota(_I32, (16, 128), 1)).astype(_F32)
    counts128 = lax.dot_general(counts, eye, (((1,), (0,)), ((), ())),
                                preferred_element_type=_F32)
    row8 = lax.broadcasted_iota(_I32, (8, 128), 0)
    meta = jnp.where(row8 == 0, jnp.broadcast_to(bexp, (8, 128)),
           jnp.where(row8 == 1, jnp.broadcast_to(bact, (8, 128)),
           jnp.where(row8 == 2, jnp.broadcast_to(counts128, (8, 128)),
           jnp.where(row8 == 3,
                     jnp.where(lane128[0:8, :] == 0, dummy, sinkf), 0.0))))
    meta_ref[...] = meta


def _run_router(x2d, router_w):
    rw16 = jnp.zeros((16, H), _F32).at[:NSLOT].set(router_w)
    return pl.pallas_call(
        _router_body,
        out_shape=[jax.ShapeDtypeStruct((T, 128), _F32),
                   jax.ShapeDtypeStruct((8, 128), _F32),
                   jax.ShapeDtypeStruct((1, 1), _F32)],
        out_specs=[pl.BlockSpec(memory_space=pltpu.VMEM),
                   pl.BlockSpec(memory_space=pltpu.VMEM),
                   pl.BlockSpec(memory_space=pltpu.SMEM)],
    )(x2d, rw16)


CSZ = 16


def _dispatch_body(x_hbm, idx1_hbm, idx2_hbm, dmy_hbm,
                   z_hbm, xd_hbm,
                   i1r, i2r, rva, rvb, rvc, rvd, dmyv, zrow,
                   sem, semw):
    wid = lax.axis_index("s") * NC + lax.axis_index("c")
    t0 = wid * TPW
    nch = TPW // CSZ
    pltpu.sync_copy(idx1_hbm.at[pl.ds(wid * nch, nch)], i1r)
    pltpu.sync_copy(idx2_hbm.at[pl.ds(wid * nch, nch)], i2r)
    rvs = (rva, rvb, rvc, rvd)
    pend = []
    for j in range(nch):
        pltpu.sync_copy(x_hbm.at[pl.ds(t0 + j * CSZ, CSZ)], rvs[j])
        pend.append(pltpu.async_copy(rvs[j], xd_hbm.at[i1r.at[j]], sem))
        pend.append(pltpu.async_copy(rvs[j], xd_hbm.at[i2r.at[j]], sem))
    for cp in pend:
        cp.wait()

    @pl.when(wid == 0)
    def _dummy():
        pltpu.sync_copy(dmy_hbm, dmyv)
        pltpu.sync_copy(z_hbm, zrow)
        pltpu.async_copy(zrow, xd_hbm.at[dmyv], semw).wait()


def _run_dispatch(x2d, idx1, idx2, dmy):
    mesh = plsc.VectorSubcoreMesh(core_axis_name="c", subcore_axis_name="s",
                                  num_cores=NC, num_subcores=NS)
    f = pl.kernel(
        _dispatch_body,
        out_type=[jax.ShapeDtypeStruct((PPAD, H), _F32)],
        mesh=mesh,
        compiler_params=pltpu.CompilerParams(needs_layout_passes=False),
        scratch_types=[
            pltpu.VMEM((TPW // CSZ, CSZ), _I32),
            pltpu.VMEM((TPW // CSZ, CSZ), _I32),
            pltpu.VMEM((CSZ, H), _F32), pltpu.VMEM((CSZ, H), _F32),
            pltpu.VMEM((CSZ, H), _F32), pltpu.VMEM((CSZ, H), _F32),
            pltpu.VMEM((L,), _I32),
            pltpu.VMEM((L, H), _F32),
            pltpu.SemaphoreType.DMA, pltpu.SemaphoreType.DMA,
        ],
    )
    return f(x2d, idx1.reshape(T // CSZ, CSZ), idx2.reshape(T // CSZ, CSZ),
             dmy, jnp.zeros((L, H), _F32))[0]


def _gffn_body(bexp_ref, bact_ref, xd_ref, gw_ref, uw_ref, dw_ref, yd_ref):
    i = pl.program_id(0)

    @pl.when(bact_ref[i] == 1)
    def _():
        xb = xd_ref[...]
        g = lax.dot_general(xb, gw_ref[0], (((1,), (1,)), ((), ())),
                            preferred_element_type=_F32)
        u = lax.dot_general(xb, uw_ref[0], (((1,), (1,)), ((), ())),
                            preferred_element_type=_F32)
        h = g * lax.logistic(g) * u
        yd_ref[...] = lax.dot_general(h, dw_ref[0], (((1,), (1,)), ((), ())),
                                      preferred_element_type=_F32)


def _run_gffn(bexp, bact, xd, gate_w, up_w, down_w):
    grid_spec = pltpu.PrefetchScalarGridSpec(
        num_scalar_prefetch=2,
        grid=(NBR,),
        in_specs=[
            pl.BlockSpec((BLK, H), lambda i, be, ba: (i, 0)),
            pl.BlockSpec((1, FF, H), lambda i, be, ba: (be[i], 0, 0)),
            pl.BlockSpec((1, FF, H), lambda i, be, ba: (be[i], 0, 0)),
            pl.BlockSpec((1, H, FF), lambda i, be, ba: (be[i], 0, 0)),
        ],
        out_specs=pl.BlockSpec((BLK, H), lambda i, be, ba: (i, 0)),
    )
    return pl.pallas_call(
        _gffn_body,
        grid_spec=grid_spec,
        out_shape=jax.ShapeDtypeStruct((PR, H), _F32),
        compiler_params=pltpu.CompilerParams(
            vmem_limit_bytes=120 * 1024 * 1024),
    )(bexp, bact, xd, gate_w, up_w, down_w)


def _sffn_body(x_ref, gw_ref, uw_ref, dw_ref, o_ref):
    xb = x_ref[...]
    g = lax.dot_general(xb, gw_ref[...], (((1,), (1,)), ((), ())),
                        preferred_element_type=_F32)
    u = lax.dot_general(xb, uw_ref[...], (((1,), (1,)), ((), ())),
                        preferred_element_type=_F32)
    h = g * lax.logistic(g) * u
    o_ref[...] = lax.dot_general(h, dw_ref[...], (((1,), (1,)), ((), ())),
                                 preferred_element_type=_F32)


def _run_sffn(x2d, sgw, suw, sdw):
    return pl.pallas_call(
        _sffn_body,
        grid=(T // BLK,),
        in_specs=[
            pl.BlockSpec((BLK, H), lambda i: (i, 0)),
            pl.BlockSpec((FF, H), lambda i: (0, 0)),
            pl.BlockSpec((FF, H), lambda i: (0, 0)),
            pl.BlockSpec((H, FF), lambda i: (0, 0)),
        ],
        out_specs=pl.BlockSpec((BLK, H), lambda i: (i, 0)),
        out_shape=jax.ShapeDtypeStruct((T, H), _F32),
    )(x2d, sgw, suw, sdw)


def _combine_body(yds_hbm, yd_hbm, pos1_hbm, pos2_hbm, w1_hbm, w2_hbm,
                  out_hbm,
                  p1all, p2all, w1v, w2v, r1a, r2a, acca, r1b, r2b, accb,
                  sem):
    wid = lax.axis_index("s") * NC + lax.axis_index("c")
    csz = 16
    nch = TPW // csz
    pltpu.sync_copy(pos1_hbm.at[pl.ds(wid * nch, nch)], p1all)
    pltpu.sync_copy(pos2_hbm.at[pl.ds(wid * nch, nch)], p2all)
    pltpu.sync_copy(w1_hbm.at[pl.ds(wid * TPW, TPW)], w1v)
    pltpu.sync_copy(w2_hbm.at[pl.ds(wid * TPW, TPW)], w2v)
    bufs = ((r1a, r2a, acca), (r1b, r2b, accb))

    def fetch(ch, bi):
        r1, r2, acc = bufs[bi]
        t0 = wid * TPW + ch * csz
        cp1 = pltpu.async_copy(yd_hbm.at[p1all.at[ch]], r1, sem)
        cp2 = pltpu.async_copy(yd_hbm.at[p2all.at[ch]], r2, sem)
        cp3 = pltpu.async_copy(yds_hbm.at[pl.ds(t0, csz)], acc, sem)
        return (cp1, cp2, cp3)

    pend = fetch(0, 0)
    for ch in range(nch):
        bi = ch % 2
        r1, r2, acc = bufs[bi]
        for cp in pend:
            cp.wait()
        if ch + 1 < nch:
            pend = fetch(ch + 1, (ch + 1) % 2)

        wv1 = w1v[pl.ds(ch * csz, csz)]
        wv2 = w2v[pl.ds(ch * csz, csz)]
        for r in range(csz):
            ws1 = wv1[r]
            ws2 = wv2[r]

            def add_group(k, _):
                for u in range(2):
                    o = (2 * k + u) * L
                    a = (acc[r, pl.ds(o, L)] + ws1 * r1[r, pl.ds(o, L)]
                         + ws2 * r2[r, pl.ds(o, L)])
                    acc[r, pl.ds(o, L)] = a
                return 0

            lax.fori_loop(0, H // (2 * L), add_group, 0)
        t0 = wid * TPW + ch * csz
        pltpu.sync_copy(acc, out_hbm.at[pl.ds(t0, csz)])


def _run_combine(yds, yd, pos1, pos2, w1, w2):
    mesh = plsc.VectorSubcoreMesh(core_axis_name="c", subcore_axis_name="s",
                                  num_cores=NC, num_subcores=NS)
    f = pl.kernel(
        _combine_body,
        out_type=[jax.ShapeDtypeStruct((T, H), _F32)],
        mesh=mesh,
        compiler_params=pltpu.CompilerParams(needs_layout_passes=False),
        scratch_types=[
            pltpu.VMEM((TPW // 16, 16), _I32), pltpu.VMEM((TPW // 16, 16), _I32),
            pltpu.VMEM((TPW,), _F32), pltpu.VMEM((TPW,), _F32),
            pltpu.VMEM((16, H), _F32), pltpu.VMEM((16, H), _F32),
            pltpu.VMEM((16, H), _F32),
            pltpu.VMEM((16, H), _F32), pltpu.VMEM((16, H), _F32),
            pltpu.VMEM((16, H), _F32),
            pltpu.SemaphoreType.DMA,
        ],
    )
    return f(yds, yd, pos1.reshape(T // 16, 16), pos2.reshape(T // 16, 16),
             w1, w2)[0]


def kernel(hidden_states, router_w, gate_w, up_w, down_w,
           shared_gate_w, shared_up_w, shared_down_w):
    x2d = hidden_states.reshape(T, H)

    rtr, meta, z_sum = _run_router(x2d, router_w)
    w1 = rtr[:, 2]
    w2 = rtr[:, 3]
    cpos1 = rtr[:, 4].astype(_I32)
    cpos2 = rtr[:, 5].astype(_I32)
    idx1 = rtr[:, 6].astype(_I32)
    idx2 = rtr[:, 7].astype(_I32)
    bexp = meta[0, :NBR].astype(_I32)
    bact = meta[1, :NBR].astype(_I32)
    counts = meta[2, :16]
    dmy = meta[3, :16].astype(_I32)

    xd = _run_dispatch(x2d, idx1, idx2, dmy)
    yds = _run_sffn(x2d, shared_gate_w, shared_up_w, shared_down_w)
    yd = _run_gffn(bexp, bact, xd, gate_w, up_w, down_w)
    out2d = _run_combine(yds, yd, cpos1, cpos2, w1, w2)

    final_outputs = out2d.reshape(B, S, H)

    loads = jnp.concatenate([counts[:E], jnp.full((1,), float(T), _F32)])
    loads_norm = loads / (jnp.sum(loads) + 1e-6)
    ideal = 1.0 / NSLOT
    load_balance_loss = jnp.mean((loads_norm - ideal) ** 2)
    router_z_loss = z_sum[0, 0] / T
    total_aux_loss = 0.01 * load_balance_loss + 0.01 * router_z_loss
    return final_outputs, total_aux_loss
